# Initial kernel scaffold; baseline (speedup 1.0000x reference)
import jax
import jax.numpy as jnp
from jax.experimental import pallas as pl


def kernel(data_id, data, W_embed):
    raise NotImplementedError("write your pallas kernel here")


# speedup vs baseline: 5.6654x; 5.6654x over previous
import functools

import jax
import jax.numpy as jnp
from jax import lax
from jax.experimental import pallas as pl
from jax.experimental.pallas import tpu as pltpu
from jax.experimental.pallas import tpu_sc as plsc

D1 = 64
D2 = 64
OUT_D = D1 + D2
NC, NS, L = 2, 16, 16
NW = NC * NS
W = 256


def _sc_tokenize(table, idx, val):
    N = idx.shape[0]
    rows_per_tile = N // NW
    n_chunks = rows_per_tile // W
    mesh = plsc.VectorSubcoreMesh(core_axis_name="c", subcore_axis_name="s")

    @functools.partial(
        pl.kernel,
        out_type=jax.ShapeDtypeStruct((N, OUT_D), jnp.float32),
        mesh=mesh,
        scratch_types=[
            pltpu.VMEM((W,), jnp.int32),
            pltpu.VMEM((W,), jnp.float32),
            pltpu.VMEM((W, D1), jnp.float32),
            pltpu.VMEM((W, D2), jnp.float32),
            pltpu.SemaphoreType.DMA,
        ],
        compiler_params=pltpu.CompilerParams(use_tc_tiling_on_sc=False),
    )
    def k(table_hbm, idx_hbm, val_hbm, out_hbm, idx_v, val_v, emb_v, valb_v, sem):
        wid = lax.axis_index("s") * NC + lax.axis_index("c")
        tile_base = wid * rows_per_tile

        @pl.loop(0, n_chunks)
        def _chunk(c):
            base = tile_base + c * W
            pltpu.sync_copy(idx_hbm.at[pl.ds(base, W)], idx_v)
            gather = pltpu.async_copy(table_hbm.at[idx_v], emb_v, sem)
            pltpu.sync_copy(val_hbm.at[pl.ds(base, W)], val_v)

            @pl.loop(0, W // L)
            def _group(g):
                vvec = val_v[pl.ds(g * L, L)]
                for i in range(L):
                    vec = jnp.full((L,), vvec[i], jnp.float32)
                    for kk in range(D2 // L):
                        valb_v[g * L + i, pl.ds(kk * L, L)] = vec

            gather.wait()
            pltpu.sync_copy(emb_v, out_hbm.at[pl.ds(base, W), pl.ds(0, D1)])
            pltpu.sync_copy(valb_v, out_hbm.at[pl.ds(base, W), pl.ds(D1, D2)])

    return k(table, idx, val)


def kernel(data_id, data, W_embed):
    B, S = data_id.shape
    idx = data_id.reshape(-1).astype(jnp.int32)
    val = data.reshape(-1)
    out = _sc_tokenize(W_embed, idx, val)
    return out.reshape(B, S, OUT_D)


# speedup vs baseline: 6.6089x; 1.1665x over previous
import functools

import jax
import jax.numpy as jnp
from jax import lax
from jax.experimental import pallas as pl
from jax.experimental.pallas import tpu as pltpu
from jax.experimental.pallas import tpu_sc as plsc

D1 = 64
D2 = 64
OUT_D = D1 + D2
NC, NS, L = 2, 16, 16
NW = NC * NS
W = 160
NSLOT = 4


def _sc_tokenize(table, idx, val):
    N = idx.shape[0]
    rows_per_tile = N // NW
    n_chunks = rows_per_tile // W
    assert rows_per_tile % W == 0 and n_chunks % NSLOT == 0
    mesh = plsc.VectorSubcoreMesh(core_axis_name="c", subcore_axis_name="s")

    @functools.partial(
        pl.kernel,
        out_type=jax.ShapeDtypeStruct((N, OUT_D), jnp.float32),
        mesh=mesh,
        scratch_types=[
            pltpu.VMEM((NSLOT, W), jnp.int32),
            pltpu.VMEM((NSLOT, W), jnp.float32),
            pltpu.VMEM((NSLOT, W, D1), jnp.float32),
            pltpu.VMEM((NSLOT, W, D2), jnp.float32),
            pltpu.SemaphoreType.DMA((NSLOT,)),
            pltpu.SemaphoreType.DMA((NSLOT,)),
            pltpu.SemaphoreType.DMA((NSLOT,)),
        ],
        compiler_params=pltpu.CompilerParams(use_tc_tiling_on_sc=False),
    )
    def k(table_hbm, idx_hbm, val_hbm, out_hbm,
          idx_v, val_v, emb_v, valb_v, si, sg, so):
        wid = lax.axis_index("s") * NC + lax.axis_index("c")
        tile_base = wid * rows_per_tile

        def start_in(cc, q):
            base = tile_base + cc * W
            pltpu.async_copy(idx_hbm.at[pl.ds(base, W)], idx_v.at[q], si.at[q])
            pltpu.async_copy(val_hbm.at[pl.ds(base, W)], val_v.at[q], si.at[q])

        def wait_in(q):
            pltpu.make_async_copy(idx_hbm.at[pl.ds(0, W)], idx_v.at[q],
                                  si.at[q]).wait()
            pltpu.make_async_copy(val_hbm.at[pl.ds(0, W)], val_v.at[q],
                                  si.at[q]).wait()

        def start_out(cc, q):
            base = tile_base + cc * W
            pltpu.async_copy(emb_v.at[q],
                             out_hbm.at[pl.ds(base, W), pl.ds(0, D1)], so.at[q])
            pltpu.async_copy(valb_v.at[q],
                             out_hbm.at[pl.ds(base, W), pl.ds(D1, D2)], so.at[q])

        def wait_out(q):
            pltpu.make_async_copy(emb_v.at[q],
                                  out_hbm.at[pl.ds(0, W), pl.ds(0, D1)],
                                  so.at[q]).wait()
            pltpu.make_async_copy(valb_v.at[q],
                                  out_hbm.at[pl.ds(0, W), pl.ds(D1, D2)],
                                  so.at[q]).wait()

        def build_valb(q):
            @pl.loop(0, W // L)
            def _group(g):
                vvec = val_v.at[q][pl.ds(g * L, L)]
                for i in range(L):
                    vec = jnp.full((L,), vvec[i], jnp.float32)
                    for kk in range(D2 // L):
                        valb_v[q, g * L + i, pl.ds(kk * L, L)] = vec

        for q in range(NSLOT):
            start_in(q, q)

        @pl.loop(0, n_chunks, step=NSLOT)
        def _body(c):
            gathers = []
            for q in range(NSLOT):
                cc = c + q
                wait_in(q)

                @pl.when(c > 0)
                def _():
                    wait_out(q)

                gathers.append(
                    pltpu.async_copy(table_hbm.at[idx_v.at[q]], emb_v.at[q],
                                     sg.at[q]))
                build_valb(q)
            for q in range(NSLOT):
                cc = c + q
                gathers[q].wait()
                start_out(cc, q)

                @pl.when(cc + NSLOT < n_chunks)
                def _():
                    start_in(cc + NSLOT, q)

        for q in range(NSLOT):
            wait_out(q)

    return k(table, idx, val)


def kernel(data_id, data, W_embed):
    B, S = data_id.shape
    idx = data_id.reshape(-1).astype(jnp.int32)
    val = data.reshape(-1)
    out = _sc_tokenize(W_embed, idx, val)
    return out.reshape(B, S, OUT_D)
